# baseline (device time: 52743 ns/iter reference)
import jax
import jax.numpy as jnp
from jax import lax
from jax.experimental import pallas as pl
from jax.experimental.pallas import tpu as pltpu

N_DEV = 4


def kernel(t, W):
    m_per, k = t.shape
    _, n = W.shape
    mh = m_per // 2
    mq = m_per // 4
    me = m_per // 8

    def body(t_ref, w_ref, out_ref, c1s, rs1r, rs2s, rs2r, yb, wb,
             tv, wv, ov, sems_s, sems_r, lsem):
        my = lax.axis_index("i")
        p_a = my ^ 1
        p_b = 3 - my
        a_bit = my & 1
        b_bit = my // 2
        keep1 = a_bit ^ b_bit
        keep2 = b_bit
        q1 = b_bit
        q2 = a_bit
        row1 = keep1 * mq + q1 * me
        l2 = keep2 * mq + q2 * me
        row2 = mh + l2

        def hcopy(sem_idx, src, dst):
            return pltpu.make_async_copy(src, dst, lsem.at[sem_idx])

        t_regions = [
            ((1 - keep1) * mq, mq),
            (mh + (1 - keep2) * mq, mq),
            (keep1 * mq + (1 - q1) * me, me),
            (mh + keep2 * mq + (1 - q2) * me, me),
            (row1, me),
            (row2, me),
        ]
        tdma = []
        for i, (off, sz) in enumerate(t_regions):
            cp = hcopy(i, t_ref.at[pl.ds(off, sz), :], tv.at[pl.ds(off, sz), :])
            cp.start()
            tdma.append(cp)
        kh = k // 2
        wdma = hcopy(6, w_ref.at[pl.ds(0, kh), :], wv)
        wdma.start()

        barrier_sem = pltpu.get_barrier_semaphore()
        for nbr in (p_a, p_b):
            pl.semaphore_signal(
                barrier_sem, inc=1,
                device_id=(nbr,), device_id_type=pl.DeviceIdType.MESH,
            )
        tdma[0].wait()
        c1s[0, :, :] = tv[pl.ds((1 - keep1) * mq, mq), :].astype(jnp.bfloat16)
        tdma[1].wait()
        c1s[1, :, :] = tv[pl.ds(mh + (1 - keep2) * mq, mq), :].astype(
            jnp.bfloat16
        )
        pl.semaphore_wait(barrier_sem, 2)

        def xchg(sem_idx, src, dst, target):
            return pltpu.make_async_remote_copy(
                src_ref=src, dst_ref=dst,
                send_sem=sems_s.at[sem_idx], recv_sem=sems_r.at[sem_idx],
                device_id=(target,), device_id_type=pl.DeviceIdType.MESH,
            )

        r1a = xchg(0, c1s.at[0, pl.ds((1 - q1) * me, me), :],
                   rs1r.at[0, pl.ds((1 - q1) * me, me), :], p_a)
        r1a.start()
        r2a = xchg(1, c1s.at[1, pl.ds(q2 * me, me), :],
                   rs1r.at[1, pl.ds(q2 * me, me), :], p_b)
        r2a.start()
        r1b = xchg(10, c1s.at[0, pl.ds(q1 * me, me), :],
                   rs1r.at[0, pl.ds(q1 * me, me), :], p_a)
        r1b.start()
        r2b = xchg(11, c1s.at[1, pl.ds((1 - q2) * me, me), :],
                   rs1r.at[1, pl.ds((1 - q2) * me, me), :], p_b)
        r2b.start()

        wdma.wait()
        wb[pl.ds(0, kh), :] = wv[:, :].astype(jnp.bfloat16)
        wdma2 = hcopy(7, w_ref.at[pl.ds(kh, kh), :], wv)
        wdma2.start()
        wdma2.wait()
        wb[pl.ds(kh, kh), :] = wv[:, :].astype(jnp.bfloat16)

        tdma[2].wait()
        r1a.wait()
        rs2s[0, :, :] = (
            rs1r[0, pl.ds((1 - q1) * me, me), :].astype(jnp.float32)
            + tv[pl.ds(keep1 * mq + (1 - q1) * me, me), :]
        ).astype(jnp.bfloat16)
        r3 = xchg(2, rs2s.at[0], rs2r.at[0], p_b)
        r3.start()

        tdma[3].wait()
        r2a.wait()
        rs2s[1, :, :] = (
            rs1r[1, pl.ds((1 - q2) * me, me), :].astype(jnp.float32)
            + tv[pl.ds(mh + keep2 * mq + (1 - q2) * me, me), :]
        ).astype(jnp.bfloat16)
        r4 = xchg(3, rs2s.at[1], rs2r.at[1], p_a)
        r4.start()

        tdma[4].wait()
        r1b.wait()
        r3.wait()
        s1 = (
            rs2r[0, :, :].astype(jnp.float32)
            + rs1r[0, pl.ds(q1 * me, me), :].astype(jnp.float32)
            + tv[pl.ds(row1, me), :]
        )
        y1 = lax.dot_general(
            s1.astype(jnp.bfloat16), wb[:, :],
            dimension_numbers=(((1,), (0,)), ((), ())),
            preferred_element_type=jnp.float32,
        )
        ov[pl.ds(row1, me), :] = y1
        yb[0, pl.ds(row1, me), :] = y1.astype(jnp.bfloat16)
        g1 = xchg(4, yb.at[0, pl.ds(row1, me), :],
                  yb.at[0, pl.ds(row1, me), :], p_b)
        g1.start()
        g3m = xchg(6, yb.at[0, pl.ds(row1, me), :],
                   yb.at[0, pl.ds(row1, me), :], p_a)
        g3m.start()
        o1 = hcopy(8, ov.at[pl.ds(row1, me), :], out_ref.at[pl.ds(row1, me), :])
        o1.start()

        tdma[5].wait()
        r2b.wait()
        r4.wait()
        s2 = (
            rs2r[1, :, :].astype(jnp.float32)
            + rs1r[1, pl.ds(q2 * me, me), :].astype(jnp.float32)
            + tv[pl.ds(row2, me), :]
        )
        y2 = lax.dot_general(
            s2.astype(jnp.bfloat16), wb[:, :],
            dimension_numbers=(((1,), (0,)), ((), ())),
            preferred_element_type=jnp.float32,
        )
        ov[pl.ds(row2, me), :] = y2
        yb[1, pl.ds(l2, me), :] = y2.astype(jnp.bfloat16)
        g2 = xchg(5, yb.at[1, pl.ds(l2, me), :],
                  yb.at[1, pl.ds(l2, me), :], p_a)
        g2.start()
        g4m = xchg(8, yb.at[1, pl.ds(l2, me), :],
                   yb.at[1, pl.ds(l2, me), :], p_b)
        g4m.start()
        o2 = hcopy(9, ov.at[pl.ds(row2, me), :], out_ref.at[pl.ds(row2, me), :])
        o2.start()

        pq1 = keep1 * mq + (1 - q1) * me
        g1.wait()
        g3p = xchg(7, yb.at[0, pl.ds(pq1, me), :],
                   yb.at[0, pl.ds(pq1, me), :], p_a)
        g3p.start()
        ov[pl.ds(pq1, me), :] = yb[0, pl.ds(pq1, me), :].astype(jnp.float32)
        o3 = hcopy(10, ov.at[pl.ds(pq1, me), :], out_ref.at[pl.ds(pq1, me), :])
        o3.start()

        pq2 = keep2 * mq + (1 - q2) * me
        g2.wait()
        g4p = xchg(9, yb.at[1, pl.ds(pq2, me), :],
                   yb.at[1, pl.ds(pq2, me), :], p_b)
        g4p.start()
        ov[pl.ds(mh + pq2, me), :] = yb[1, pl.ds(pq2, me), :].astype(
            jnp.float32
        )
        o4 = hcopy(11, ov.at[pl.ds(mh + pq2, me), :],
                   out_ref.at[pl.ds(mh + pq2, me), :])
        o4.start()

        oq1a = (1 - keep1) * mq + q1 * me
        oq1b = (1 - keep1) * mq + (1 - q1) * me
        g3m.wait()
        ov[pl.ds(oq1a, me), :] = yb[0, pl.ds(oq1a, me), :].astype(jnp.float32)
        o5 = hcopy(12, ov.at[pl.ds(oq1a, me), :],
                   out_ref.at[pl.ds(oq1a, me), :])
        o5.start()
        oq2a = (1 - keep2) * mq + (1 - q2) * me
        oq2b = (1 - keep2) * mq + q2 * me
        g4m.wait()
        ov[pl.ds(mh + oq2a, me), :] = yb[1, pl.ds(oq2a, me), :].astype(
            jnp.float32
        )
        o6 = hcopy(13, ov.at[pl.ds(mh + oq2a, me), :],
                   out_ref.at[pl.ds(mh + oq2a, me), :])
        o6.start()
        g3p.wait()
        ov[pl.ds(oq1b, me), :] = yb[0, pl.ds(oq1b, me), :].astype(jnp.float32)
        o7 = hcopy(14, ov.at[pl.ds(oq1b, me), :],
                   out_ref.at[pl.ds(oq1b, me), :])
        o7.start()
        g4p.wait()
        ov[pl.ds(mh + oq2b, me), :] = yb[1, pl.ds(oq2b, me), :].astype(
            jnp.float32
        )
        o8 = hcopy(15, ov.at[pl.ds(mh + oq2b, me), :],
                   out_ref.at[pl.ds(mh + oq2b, me), :])
        o8.start()

        for cp in (o1, o2, o3, o4, o5, o6, o7, o8):
            cp.wait()

    return pl.pallas_call(
        body,
        out_shape=jax.ShapeDtypeStruct((m_per, n), jnp.float32),
        in_specs=[
            pl.BlockSpec(memory_space=pl.ANY),
            pl.BlockSpec(memory_space=pl.ANY),
        ],
        out_specs=pl.BlockSpec(memory_space=pl.ANY),
        scratch_shapes=[
            pltpu.VMEM((2, mq, k), jnp.bfloat16),
            pltpu.VMEM((2, mq, k), jnp.bfloat16),
            pltpu.VMEM((2, me, k), jnp.bfloat16),
            pltpu.VMEM((2, me, k), jnp.bfloat16),
            pltpu.VMEM((2, mh, n), jnp.bfloat16),
            pltpu.VMEM((k, n), jnp.bfloat16),
            pltpu.VMEM((m_per, k), jnp.float32),
            pltpu.VMEM((k // 2, n), jnp.float32),
            pltpu.VMEM((m_per, n), jnp.float32),
            pltpu.SemaphoreType.DMA((12,)),
            pltpu.SemaphoreType.DMA((12,)),
            pltpu.SemaphoreType.DMA((16,)),
        ],
        compiler_params=pltpu.CompilerParams(collective_id=0),
    )(t, W)


# device time: 51729 ns/iter; 1.0196x vs baseline; 1.0196x over previous
import jax
import jax.numpy as jnp
from jax import lax
from jax.experimental import pallas as pl
from jax.experimental.pallas import tpu as pltpu

N_DEV = 4


def kernel(t, W):
    m_per, k = t.shape
    _, n = W.shape
    mh = m_per // 2
    mq = m_per // 4
    me = m_per // 8

    def body(t_ref, w_ref, out_ref, c1s, rs1r, rs2s, rs2r, yb, wb, sems_s, sems_r):
        my = lax.axis_index("i")
        p_a = my ^ 1
        p_b = 3 - my
        a_bit = my & 1
        b_bit = my // 2
        keep1 = a_bit ^ b_bit
        keep2 = b_bit
        q1 = b_bit
        q2 = a_bit
        row1 = keep1 * mq + q1 * me
        l2 = keep2 * mq + q2 * me
        row2 = mh + l2

        barrier_sem = pltpu.get_barrier_semaphore()
        for nbr in (p_a, p_b):
            pl.semaphore_signal(
                barrier_sem, inc=1,
                device_id=(nbr,), device_id_type=pl.DeviceIdType.MESH,
            )
        c1s[0, :, :] = t_ref[pl.ds((1 - keep1) * mq, mq), :].astype(jnp.bfloat16)
        c1s[1, :, :] = t_ref[pl.ds(mh + (1 - keep2) * mq, mq), :].astype(
            jnp.bfloat16
        )
        wb[:, :] = w_ref[:, :].astype(jnp.bfloat16)
        pl.semaphore_wait(barrier_sem, 2)

        def xchg(sem_idx, src, dst, target):
            return pltpu.make_async_remote_copy(
                src_ref=src, dst_ref=dst,
                send_sem=sems_s.at[sem_idx], recv_sem=sems_r.at[sem_idx],
                device_id=(target,), device_id_type=pl.DeviceIdType.MESH,
            )

        r1a = xchg(0, c1s.at[0, pl.ds((1 - q1) * me, me), :],
                   rs1r.at[0, pl.ds((1 - q1) * me, me), :], p_a)
        r1a.start()
        r2a = xchg(1, c1s.at[1, pl.ds(q2 * me, me), :],
                   rs1r.at[1, pl.ds(q2 * me, me), :], p_b)
        r2a.start()
        r1b = xchg(10, c1s.at[0, pl.ds(q1 * me, me), :],
                   rs1r.at[0, pl.ds(q1 * me, me), :], p_a)
        r1b.start()
        r2b = xchg(11, c1s.at[1, pl.ds((1 - q2) * me, me), :],
                   rs1r.at[1, pl.ds((1 - q2) * me, me), :], p_b)
        r2b.start()

        r1a.wait()
        rs2s[0, :, :] = (
            rs1r[0, pl.ds((1 - q1) * me, me), :].astype(jnp.float32)
            + t_ref[pl.ds(keep1 * mq + (1 - q1) * me, me), :]
        ).astype(jnp.bfloat16)
        r3 = xchg(2, rs2s.at[0], rs2r.at[0], p_b)
        r3.start()

        r2a.wait()
        rs2s[1, :, :] = (
            rs1r[1, pl.ds((1 - q2) * me, me), :].astype(jnp.float32)
            + t_ref[pl.ds(mh + keep2 * mq + (1 - q2) * me, me), :]
        ).astype(jnp.bfloat16)
        r4 = xchg(3, rs2s.at[1], rs2r.at[1], p_a)
        r4.start()

        r1b.wait()
        r3.wait()
        s1 = (
            rs2r[0, :, :].astype(jnp.float32)
            + rs1r[0, pl.ds(q1 * me, me), :].astype(jnp.float32)
            + t_ref[pl.ds(row1, me), :]
        )
        y1 = lax.dot_general(
            s1.astype(jnp.bfloat16), wb[:, :],
            dimension_numbers=(((1,), (0,)), ((), ())),
            preferred_element_type=jnp.float32,
        )
        yb[0, pl.ds(row1, me), :] = y1.astype(jnp.bfloat16)
        g1 = xchg(4, yb.at[0, pl.ds(row1, me), :],
                  yb.at[0, pl.ds(row1, me), :], p_b)
        g1.start()
        g3m = xchg(6, yb.at[0, pl.ds(row1, me), :],
                   yb.at[0, pl.ds(row1, me), :], p_a)
        g3m.start()
        out_ref[pl.ds(row1, me), :] = y1

        r2b.wait()
        r4.wait()
        s2 = (
            rs2r[1, :, :].astype(jnp.float32)
            + rs1r[1, pl.ds(q2 * me, me), :].astype(jnp.float32)
            + t_ref[pl.ds(row2, me), :]
        )
        y2 = lax.dot_general(
            s2.astype(jnp.bfloat16), wb[:, :],
            dimension_numbers=(((1,), (0,)), ((), ())),
            preferred_element_type=jnp.float32,
        )
        yb[1, pl.ds(l2, me), :] = y2.astype(jnp.bfloat16)
        g2 = xchg(5, yb.at[1, pl.ds(l2, me), :],
                  yb.at[1, pl.ds(l2, me), :], p_a)
        g2.start()
        g4m = xchg(8, yb.at[1, pl.ds(l2, me), :],
                   yb.at[1, pl.ds(l2, me), :], p_b)
        g4m.start()
        out_ref[pl.ds(row2, me), :] = y2

        pq1 = keep1 * mq + (1 - q1) * me
        g1.wait()
        g3p = xchg(7, yb.at[0, pl.ds(pq1, me), :],
                   yb.at[0, pl.ds(pq1, me), :], p_a)
        g3p.start()
        out_ref[pl.ds(pq1, me), :] = yb[0, pl.ds(pq1, me), :].astype(jnp.float32)

        pq2 = keep2 * mq + (1 - q2) * me
        g2.wait()
        g4p = xchg(9, yb.at[1, pl.ds(pq2, me), :],
                   yb.at[1, pl.ds(pq2, me), :], p_b)
        g4p.start()
        out_ref[pl.ds(mh + pq2, me), :] = yb[1, pl.ds(pq2, me), :].astype(
            jnp.float32
        )

        oq1a = (1 - keep1) * mq + q1 * me
        oq1b = (1 - keep1) * mq + (1 - q1) * me
        g3m.wait()
        out_ref[pl.ds(oq1a, me), :] = yb[0, pl.ds(oq1a, me), :].astype(
            jnp.float32
        )
        oq2a = (1 - keep2) * mq + (1 - q2) * me
        oq2b = (1 - keep2) * mq + q2 * me
        g4m.wait()
        out_ref[pl.ds(mh + oq2a, me), :] = yb[1, pl.ds(oq2a, me), :].astype(
            jnp.float32
        )
        g3p.wait()
        out_ref[pl.ds(oq1b, me), :] = yb[0, pl.ds(oq1b, me), :].astype(
            jnp.float32
        )
        g4p.wait()
        out_ref[pl.ds(mh + oq2b, me), :] = yb[1, pl.ds(oq2b, me), :].astype(
            jnp.float32
        )

    return pl.pallas_call(
        body,
        out_shape=jax.ShapeDtypeStruct((m_per, n), jnp.float32),
        in_specs=[
            pl.BlockSpec(memory_space=pltpu.VMEM),
            pl.BlockSpec(memory_space=pltpu.VMEM),
        ],
        out_specs=pl.BlockSpec(memory_space=pltpu.VMEM),
        scratch_shapes=[
            pltpu.VMEM((2, mq, k), jnp.bfloat16),
            pltpu.VMEM((2, mq, k), jnp.bfloat16),
            pltpu.VMEM((2, me, k), jnp.bfloat16),
            pltpu.VMEM((2, me, k), jnp.bfloat16),
            pltpu.VMEM((2, mh, n), jnp.bfloat16),
            pltpu.VMEM((k, n), jnp.bfloat16),
            pltpu.SemaphoreType.DMA((12,)),
            pltpu.SemaphoreType.DMA((12,)),
        ],
        compiler_params=pltpu.CompilerParams(collective_id=0),
    )(t, W)
